# Initial kernel scaffold; baseline (speedup 1.0000x reference)
import functools

import jax
import jax.numpy as jnp
from jax import lax
from jax.experimental import pallas as pl
from jax.experimental.pallas import tpu as pltpu
from jax.experimental.pallas import tpu_sc as plsc

N_NODES = 10000
N_EDGES = 320000
D = 128

NC = 2
NS = 16
K = 80
NCHUNK = 125
ROWS_PER_TEC = N_NODES // NS
ZCHUNK = 125
NZ = ROWS_PER_TEC // ZCHUNK
CW = 16


def _sc_agg_body(with_cnt, *refs):
    if with_cnt:
        (x_hbm, src_hbm, dst_hbm, z128, z16, agg_out, cnt_out,
         idx_src, idx_dst, rows, ones_v, sh_agg, sh_cnt) = refs
    else:
        (x_hbm, src_hbm, dst_hbm, z128, agg_out,
         idx_src, idx_dst, rows, sh_agg) = refs

    c = lax.axis_index("c")
    s = lax.axis_index("s")

    for k in range(NZ):
        row0 = (s * NZ + k) * ZCHUNK
        pltpu.sync_copy(z128, sh_agg.at[pl.ds(row0, ZCHUNK)])
        if with_cnt:
            pltpu.sync_copy(z16, sh_cnt.at[pl.ds(row0, ZCHUNK)])

    pltpu.sync_copy(src_hbm.at[c, s], idx_src)
    pltpu.sync_copy(dst_hbm.at[c, s], idx_dst)

    if with_cnt:
        def fill_ones(i, carry):
            ones_v[i, :] = jnp.ones((CW,), jnp.float32)
            return carry
        lax.fori_loop(0, K, fill_ones, 0)

    plsc.subcore_barrier()

    def chunk(j, carry):
        pltpu.sync_copy(x_hbm.at[idx_src.at[j]], rows)
        pltpu.sync_copy(rows, sh_agg.at[idx_dst.at[j]], add=True)
        if with_cnt:
            pltpu.sync_copy(ones_v, sh_cnt.at[idx_dst.at[j]], add=True)
        return carry
    lax.fori_loop(0, NCHUNK, chunk, 0)

    plsc.subcore_barrier()

    for k in range(NZ):
        row0 = (s * NZ + k) * ZCHUNK
        pltpu.sync_copy(sh_agg.at[pl.ds(row0, ZCHUNK)],
                        agg_out.at[c, pl.ds(row0, ZCHUNK)])
        if with_cnt:
            pltpu.sync_copy(sh_cnt.at[pl.ds(row0, ZCHUNK)],
                            cnt_out.at[c, pl.ds(row0, ZCHUNK)])


def _make_sc_agg(with_cnt):
    mesh = plsc.VectorSubcoreMesh(core_axis_name="c", subcore_axis_name="s",
                                  num_cores=NC, num_subcores=NS)
    out_type = [jax.ShapeDtypeStruct((NC, N_NODES, D), jnp.float32)]
    scratch = [
        pltpu.VMEM((NCHUNK, K), jnp.int32),
        pltpu.VMEM((NCHUNK, K), jnp.int32),
        pltpu.VMEM((K, D), jnp.float32),
    ]
    if with_cnt:
        out_type.append(jax.ShapeDtypeStruct((NC, N_NODES, CW), jnp.float32))
        scratch.append(pltpu.VMEM((K, CW), jnp.float32))
    scratch.append(pltpu.VMEM_SHARED((N_NODES, D), jnp.float32))
    if with_cnt:
        scratch.append(pltpu.VMEM_SHARED((N_NODES, CW), jnp.float32))
    return pl.kernel(functools.partial(_sc_agg_body, with_cnt),
                     out_type=out_type, mesh=mesh, scratch_types=scratch,
                     name="sc_sage_agg_cnt" if with_cnt else "sc_sage_agg")


_sc_agg_cnt = _make_sc_agg(True)
_sc_agg = _make_sc_agg(False)


def _tc_update_body(relu, agg_ref, cnt_ref, x_ref, wl_ref, wr_ref, b_ref,
                    o_ref):
    cnt = cnt_ref[0] + cnt_ref[1]
    rcp = 1.0 / jnp.maximum(cnt[:, 0:1], 1.0)
    mean = (agg_ref[0] + agg_ref[1]) * rcp
    acc = jnp.dot(mean, wl_ref[...], preferred_element_type=jnp.float32)
    acc = acc + jnp.dot(x_ref[...], wr_ref[...],
                        preferred_element_type=jnp.float32)
    acc = acc + b_ref[...]
    if relu:
        acc = jnp.maximum(acc, 0.0)
    o_ref[...] = acc


def _make_tc_update(relu, rows=400):
    grid = N_NODES // rows
    return pl.pallas_call(
        functools.partial(_tc_update_body, relu),
        grid=(grid,),
        in_specs=[
            pl.BlockSpec((NC, rows, D), lambda i: (0, i, 0)),
            pl.BlockSpec((NC, rows, CW), lambda i: (0, i, 0)),
            pl.BlockSpec((rows, D), lambda i: (i, 0)),
            pl.BlockSpec((D, D), lambda i: (0, 0)),
            pl.BlockSpec((D, D), lambda i: (0, 0)),
            pl.BlockSpec((1, D), lambda i: (0, 0)),
        ],
        out_specs=pl.BlockSpec((rows, D), lambda i: (i, 0)),
        out_shape=jax.ShapeDtypeStruct((N_NODES, D), jnp.float32),
        name="tc_sage_update_relu" if relu else "tc_sage_update",
    )


_tc_update_relu = _make_tc_update(True)
_tc_update = _make_tc_update(False)


def kernel(x, edge_index, Wl0, Wr0, b0, Wl1, Wr1, b1):
    src = edge_index[0].astype(jnp.int32).reshape(NC, NS, NCHUNK, K)
    dst = edge_index[1].astype(jnp.int32).reshape(NC, NS, NCHUNK, K)
    z128 = jnp.zeros((ZCHUNK, D), jnp.float32)
    z16 = jnp.zeros((ZCHUNK, CW), jnp.float32)

    aggp, cntp = _sc_agg_cnt(x, src, dst, z128, z16)
    h1 = _tc_update_relu(aggp, cntp, x, Wl0, Wr0, b0.reshape(1, D))
    aggp2, = _sc_agg(h1, src, dst, z128)
    out = _tc_update(aggp2, cntp, h1, Wl1, Wr1, b1.reshape(1, D))
    return out


# speedup vs baseline: 6.4172x; 6.4172x over previous
import functools

import jax
import jax.numpy as jnp
from jax import lax
from jax.experimental import pallas as pl
from jax.experimental.pallas import tpu as pltpu
from jax.experimental.pallas import tpu_sc as plsc

N_NODES = 10000
N_EDGES = 320000
D = 128

NC = 2
NS = 16
K = 80
NCHUNK = 125
ZCHUNK = 200
NCOPY = N_NODES // ZCHUNK
KMAX = -(-NCOPY // NS)
CW = 128


def _sc_agg_body(x_hbm, src_hbm, dst_hbm, z128, agg_out,
                 idx_src, idx_dst, rows, sh_agg):
    c = lax.axis_index("c")
    s = lax.axis_index("s")

    for k in range(KMAX):
        cid = s + k * NS
        row0 = cid * ZCHUNK

        @pl.when(cid < NCOPY)
        def _():
            pltpu.sync_copy(z128, sh_agg.at[pl.ds(row0, ZCHUNK)])

    pltpu.sync_copy(src_hbm.at[c, s], idx_src)
    pltpu.sync_copy(dst_hbm.at[c, s], idx_dst)

    plsc.subcore_barrier()

    def chunk(j, carry):
        pltpu.sync_copy(x_hbm.at[idx_src.at[j]], rows)
        pltpu.sync_copy(rows, sh_agg.at[idx_dst.at[j]], add=True)
        return carry
    lax.fori_loop(0, NCHUNK, chunk, 0)

    plsc.subcore_barrier()

    for k in range(KMAX):
        cid = s + k * NS
        row0 = cid * ZCHUNK

        @pl.when(cid < NCOPY)
        def _():
            pltpu.sync_copy(sh_agg.at[pl.ds(row0, ZCHUNK)],
                            agg_out.at[c, pl.ds(row0, ZCHUNK)])


def _sc_cnt_body(dst_hbm, z16, ones_hbm, cnt_out, idx_dst, ones_v, sh_cnt):
    c = lax.axis_index("c")
    s = lax.axis_index("s")

    for k in range(KMAX):
        cid = s + k * NS
        row0 = cid * ZCHUNK

        @pl.when(cid < NCOPY)
        def _():
            pltpu.sync_copy(z16, sh_cnt.at[pl.ds(row0, ZCHUNK)])

    pltpu.sync_copy(dst_hbm.at[c, s], idx_dst)
    pltpu.sync_copy(ones_hbm, ones_v)

    plsc.subcore_barrier()

    def chunk(j, carry):
        pltpu.sync_copy(ones_v, sh_cnt.at[idx_dst.at[j]], add=True)
        return carry
    lax.fori_loop(0, NCHUNK, chunk, 0)

    plsc.subcore_barrier()

    for k in range(KMAX):
        cid = s + k * NS
        row0 = cid * ZCHUNK

        @pl.when(cid < NCOPY)
        def _():
            pltpu.sync_copy(sh_cnt.at[pl.ds(row0, ZCHUNK)],
                            cnt_out.at[c, pl.ds(row0, ZCHUNK)])


_sc_mesh = plsc.VectorSubcoreMesh(core_axis_name="c", subcore_axis_name="s",
                                  num_cores=NC, num_subcores=NS)

_sc_agg = pl.kernel(
    _sc_agg_body,
    out_type=[jax.ShapeDtypeStruct((NC, N_NODES, D), jnp.float32)],
    mesh=_sc_mesh,
    scratch_types=[
        pltpu.VMEM((NCHUNK, K), jnp.int32),
        pltpu.VMEM((NCHUNK, K), jnp.int32),
        pltpu.VMEM((K, D), jnp.float32),
        pltpu.VMEM_SHARED((N_NODES, D), jnp.float32),
    ],
    name="sc_sage_agg")

_sc_cnt = pl.kernel(
    _sc_cnt_body,
    out_type=[jax.ShapeDtypeStruct((NC, N_NODES, CW), jnp.float32)],
    mesh=_sc_mesh,
    scratch_types=[
        pltpu.VMEM((NCHUNK, K), jnp.int32),
        pltpu.VMEM((K, CW), jnp.float32),
        pltpu.VMEM_SHARED((N_NODES, CW), jnp.float32),
    ],
    name="sc_sage_cnt")


def _tc_update_body(relu, agg_ref, cnt_ref, x_ref, wl_ref, wr_ref, b_ref,
                    o_ref):
    cnt = cnt_ref[0] + cnt_ref[1]
    rcp = 1.0 / jnp.maximum(cnt[:, 0:1], 1.0)
    mean = (agg_ref[0] + agg_ref[1]) * rcp
    acc = jnp.dot(mean, wl_ref[...], preferred_element_type=jnp.float32,
                  precision=lax.Precision.HIGHEST)
    acc = acc + jnp.dot(x_ref[...], wr_ref[...],
                        preferred_element_type=jnp.float32,
                        precision=lax.Precision.HIGHEST)
    acc = acc + b_ref[...]
    if relu:
        acc = jnp.maximum(acc, 0.0)
    o_ref[...] = acc


def _make_tc_update(relu, rows=400):
    grid = N_NODES // rows
    return pl.pallas_call(
        functools.partial(_tc_update_body, relu),
        grid=(grid,),
        in_specs=[
            pl.BlockSpec((NC, rows, D), lambda i: (0, i, 0)),
            pl.BlockSpec((NC, rows, CW), lambda i: (0, i, 0)),
            pl.BlockSpec((rows, D), lambda i: (i, 0)),
            pl.BlockSpec((D, D), lambda i: (0, 0)),
            pl.BlockSpec((D, D), lambda i: (0, 0)),
            pl.BlockSpec((1, D), lambda i: (0, 0)),
        ],
        out_specs=pl.BlockSpec((rows, D), lambda i: (i, 0)),
        out_shape=jax.ShapeDtypeStruct((N_NODES, D), jnp.float32),
        name="tc_sage_update_relu" if relu else "tc_sage_update",
    )


_tc_update_relu = _make_tc_update(True)
_tc_update = _make_tc_update(False)


def kernel(x, edge_index, Wl0, Wr0, b0, Wl1, Wr1, b1):
    src = edge_index[0].astype(jnp.int32).reshape(NC, NS, NCHUNK, K)
    dst = edge_index[1].astype(jnp.int32).reshape(NC, NS, NCHUNK, K)
    z128 = jnp.zeros((ZCHUNK, D), jnp.float32)
    z16 = jnp.zeros((ZCHUNK, CW), jnp.float32)
    ones = jnp.ones((K, CW), jnp.float32)

    cntp, = _sc_cnt(dst, z16, ones)
    aggp, = _sc_agg(x, src, dst, z128)
    h1 = _tc_update_relu(aggp, cntp, x, Wl0, Wr0, b0.reshape(1, D))
    aggp2, = _sc_agg(h1, src, dst, z128)
    out = _tc_update(aggp2, cntp, h1, Wl1, Wr1, b1.reshape(1, D))
    return out


# speedup vs baseline: 10.2034x; 1.5900x over previous
import functools

import jax
import jax.numpy as jnp
from jax import lax
from jax.experimental import pallas as pl
from jax.experimental.pallas import tpu as pltpu
from jax.experimental.pallas import tpu_sc as plsc

N_NODES = 10000
N_EDGES = 320000
D = 128

NC = 2
NS = 16
K = 40
NCHUNK = 250
PC = 25
P = NCHUNK // PC
ZCHUNK = 200
NCOPY = N_NODES // ZCHUNK
KMAX = -(-NCOPY // NS)
CW = 128
NB = 5


def _sc_agg_body(x_hbm, src_hbm, dst_hbm, z128, agg_out, *rest):
    rows = rest[:NB]
    isrc = rest[NB:NB + 2]
    idst = rest[NB + 2:NB + 4]
    sh_agg = rest[NB + 4]
    gsem = rest[NB + 5:2 * NB + 5]
    isem = rest[2 * NB + 5:]
    c = lax.axis_index("c")
    s = lax.axis_index("s")

    pltpu.sync_copy(src_hbm.at[c, s, 0], isrc[0])
    pltpu.sync_copy(dst_hbm.at[c, s, 0], idst[0])
    for b in range(NB):
        pltpu.async_copy(x_hbm.at[isrc[0].at[b]], rows[b], gsem[b])
    pltpu.async_copy(src_hbm.at[c, s, 1], isrc[1], isem[1])
    pltpu.async_copy(dst_hbm.at[c, s, 1], idst[1], isem[1])

    for k in range(KMAX):
        cid = s + k * NS
        row0 = cid * ZCHUNK

        @pl.when(cid < NCOPY)
        def _():
            pltpu.sync_copy(z128, sh_agg.at[pl.ds(row0, ZCHUNK)])

    plsc.subcore_barrier()

    for p in range(P):
        pp = p % 2
        if p >= 1 and p + 1 < P:
            pltpu.async_copy(src_hbm.at[c, s, p + 1], isrc[1 - pp],
                             isem[1 - pp])
            pltpu.async_copy(dst_hbm.at[c, s, p + 1], idst[1 - pp],
                             isem[1 - pp])

        def grp(g, cc, pp=pp):
            j0 = g * NB
            for b in range(NB):
                j = j0 + b
                pltpu.make_async_copy(x_hbm.at[isrc[pp].at[j]], rows[b],
                                      gsem[b]).wait()
                pltpu.sync_copy(rows[b], sh_agg.at[idst[pp].at[j]], add=True)
                pltpu.async_copy(x_hbm.at[isrc[pp].at[j + NB]], rows[b],
                                 gsem[b])
            return cc
        lax.fori_loop(0, PC // NB - 1, grp, 0)

        if p + 1 < P:
            pltpu.make_async_copy(src_hbm.at[c, s, p + 1], isrc[1 - pp],
                                  isem[1 - pp]).wait()
            pltpu.make_async_copy(dst_hbm.at[c, s, p + 1], idst[1 - pp],
                                  isem[1 - pp]).wait()

        for b in range(NB):
            j = PC - NB + b
            pltpu.make_async_copy(x_hbm.at[isrc[pp].at[j]], rows[b],
                                  gsem[b]).wait()
            pltpu.sync_copy(rows[b], sh_agg.at[idst[pp].at[j]], add=True)
            if p + 1 < P:
                pltpu.async_copy(x_hbm.at[isrc[1 - pp].at[b]], rows[b],
                                 gsem[b])

    plsc.subcore_barrier()

    for k in range(KMAX):
        cid = s + k * NS
        row0 = cid * ZCHUNK

        @pl.when(cid < NCOPY)
        def _():
            pltpu.sync_copy(sh_agg.at[pl.ds(row0, ZCHUNK)],
                            agg_out.at[c, pl.ds(row0, ZCHUNK)])


def _sc_cnt_body(dst_hbm, z16, ones_hbm, cnt_out, idx_dst, ones_v, sh_cnt,
                 *ssem):
    c = lax.axis_index("c")
    s = lax.axis_index("s")

    pltpu.sync_copy(ones_hbm, ones_v)

    for k in range(KMAX):
        cid = s + k * NS
        row0 = cid * ZCHUNK

        @pl.when(cid < NCOPY)
        def _():
            pltpu.sync_copy(z16, sh_cnt.at[pl.ds(row0, ZCHUNK)])

    plsc.subcore_barrier()

    def phase(p, carry):
        pltpu.sync_copy(dst_hbm.at[c, s, p], idx_dst)

        def grp(g, cc):
            j0 = g * NB
            for b in range(NB):
                pltpu.async_copy(ones_v, sh_cnt.at[idx_dst.at[j0 + b]],
                                 ssem[b], add=True)
            for b in range(NB):
                pltpu.make_async_copy(ones_v, sh_cnt.at[idx_dst.at[j0 + b]],
                                      ssem[b]).wait()
            return cc
        lax.fori_loop(0, PC // NB, grp, 0)
        return carry
    lax.fori_loop(0, P, phase, 0)

    plsc.subcore_barrier()

    for k in range(KMAX):
        cid = s + k * NS
        row0 = cid * ZCHUNK

        @pl.when(cid < NCOPY)
        def _():
            pltpu.sync_copy(sh_cnt.at[pl.ds(row0, ZCHUNK)],
                            cnt_out.at[c, pl.ds(row0, ZCHUNK)])


_sc_mesh = plsc.VectorSubcoreMesh(core_axis_name="c", subcore_axis_name="s",
                                  num_cores=NC, num_subcores=NS)

_sc_agg = pl.kernel(
    _sc_agg_body,
    out_type=[jax.ShapeDtypeStruct((NC, N_NODES, D), jnp.float32)],
    mesh=_sc_mesh,
    scratch_types=(
        [pltpu.VMEM((K, D), jnp.float32)] * NB
        + [pltpu.VMEM((PC, K), jnp.int32)] * 4
        + [pltpu.VMEM_SHARED((N_NODES, D), jnp.float32)]
        + [pltpu.SemaphoreType.DMA] * (NB + 2)
    ),
    name="sc_sage_agg")

_sc_cnt = pl.kernel(
    _sc_cnt_body,
    out_type=[jax.ShapeDtypeStruct((NC, N_NODES, CW), jnp.float32)],
    mesh=_sc_mesh,
    scratch_types=(
        [pltpu.VMEM((PC, K), jnp.int32),
         pltpu.VMEM((K, CW), jnp.float32),
         pltpu.VMEM_SHARED((N_NODES, CW), jnp.float32)]
        + [pltpu.SemaphoreType.DMA] * NB
    ),
    name="sc_sage_cnt")


def _tc_update_body(relu, agg_ref, cnt_ref, x_ref, wl_ref, wr_ref, b_ref,
                    o_ref):
    cnt = cnt_ref[0] + cnt_ref[1]
    rcp = 1.0 / jnp.maximum(cnt[:, 0:1], 1.0)
    mean = (agg_ref[0] + agg_ref[1]) * rcp
    acc = jnp.dot(mean, wl_ref[...], preferred_element_type=jnp.float32,
                  precision=lax.Precision.HIGHEST)
    acc = acc + jnp.dot(x_ref[...], wr_ref[...],
                        preferred_element_type=jnp.float32,
                        precision=lax.Precision.HIGHEST)
    acc = acc + b_ref[...]
    if relu:
        acc = jnp.maximum(acc, 0.0)
    o_ref[...] = acc


def _make_tc_update(relu, rows=400):
    grid = N_NODES // rows
    return pl.pallas_call(
        functools.partial(_tc_update_body, relu),
        grid=(grid,),
        in_specs=[
            pl.BlockSpec((NC, rows, D), lambda i: (0, i, 0)),
            pl.BlockSpec((NC, rows, CW), lambda i: (0, i, 0)),
            pl.BlockSpec((rows, D), lambda i: (i, 0)),
            pl.BlockSpec((D, D), lambda i: (0, 0)),
            pl.BlockSpec((D, D), lambda i: (0, 0)),
            pl.BlockSpec((1, D), lambda i: (0, 0)),
        ],
        out_specs=pl.BlockSpec((rows, D), lambda i: (i, 0)),
        out_shape=jax.ShapeDtypeStruct((N_NODES, D), jnp.float32),
        name="tc_sage_update_relu" if relu else "tc_sage_update",
    )


_tc_update_relu = _make_tc_update(True)
_tc_update = _make_tc_update(False)


def kernel(x, edge_index, Wl0, Wr0, b0, Wl1, Wr1, b1):
    src = edge_index[0].astype(jnp.int32).reshape(NC, NS, P, PC, K)
    dst = edge_index[1].astype(jnp.int32).reshape(NC, NS, P, PC, K)
    z128 = jnp.zeros((ZCHUNK, D), jnp.float32)
    z16 = jnp.zeros((ZCHUNK, CW), jnp.float32)
    ones = jnp.ones((K, CW), jnp.float32)

    cntp, = _sc_cnt(dst, z16, ones)
    aggp, = _sc_agg(x, src, dst, z128)
    h1 = _tc_update_relu(aggp, cntp, x, Wl0, Wr0, b0.reshape(1, D))
    aggp2, = _sc_agg(h1, src, dst, z128)
    out = _tc_update(aggp2, cntp, h1, Wl1, Wr1, b1.reshape(1, D))
    return out
